# baseline (speedup 1.0000x reference)
import functools

import jax
import jax.numpy as jnp
from jax import lax
from jax.experimental import pallas as pl
from jax.experimental.pallas import tpu as pltpu
from jax.experimental.pallas import tpu_sc as plsc

EMBED = 32
LANES = 16
WIN = 256
WSHIFT = 8
RING = 16
ROWPAD = 128
NB = 128
BBITS = 14


def kernel(users, items, user_table, item_table):
    users = users.astype(jnp.int32)
    items = items.astype(jnp.int32)
    ut = user_table.T
    it = item_table.T
    batch = users.shape[0]
    n = user_table.shape[0]

    nfull = n // WIN
    tail = n - nfull * WIN

    info = plsc.get_sparse_core_info()
    nc, ns = info.num_cores, info.num_subcores
    nw = nc * ns
    bpw = batch // nw
    wper = nfull // nw
    wextra = nfull - wper * nw
    brows = batch + 8

    mesh = plsc.VectorSubcoreMesh(core_axis_name="c", subcore_axis_name="s")
    lanes = lambda: lax.iota(jnp.int32, LANES)

    @functools.partial(
        pl.kernel,
        mesh=mesh,
        compiler_params=pltpu.CompilerParams(needs_layout_passes=False),
        out_type=(jax.ShapeDtypeStruct((brows, ROWPAD), jnp.float32),
                  jax.ShapeDtypeStruct((brows, ROWPAD), jnp.float32)),
        scratch_types=[
            pltpu.VMEM((batch,), jnp.int32),
            pltpu.VMEM((batch,), jnp.int32),
            pltpu.VMEM((2, batch), jnp.int32),
            pltpu.VMEM((2, NB), jnp.int32),
            pltpu.VMEM((LANES, NB), jnp.int32),
            pltpu.VMEM((LANES, NB), jnp.int32),
            pltpu.VMEM((2, EMBED, WIN), jnp.float32),
            pltpu.VMEM((2, EMBED, WIN), jnp.float32),
            pltpu.VMEM((2, RING, ROWPAD), jnp.float32),
            pltpu.VMEM((2, RING, ROWPAD), jnp.float32),
            pltpu.VMEM((2, RING), jnp.int32),
            pltpu.VMEM((2, RING), jnp.int32),
            pltpu.VMEM((EMBED, tail or 1), jnp.float32),
            pltpu.VMEM((EMBED, tail or 1), jnp.float32),
            pltpu.SemaphoreType.DMA((2,)),
            pltpu.SemaphoreType.DMA,
            pltpu.SemaphoreType.DMA,
        ],
    )
    def scan(users_hbm, items_hbm, ut_hbm, it_hbm, urows_hbm, irows_hbm,
             stage, raw, shits, starts, cnts, lpfx, uwin, iwin, uring, iring,
             uridx, iridx, utail, itail, wsem, fsemu, fsemi):
        wid = lax.axis_index("s") * nc + lax.axis_index("c")
        lo = wid * wper + jnp.minimum(wid, wextra)
        cnt_w = wper + jnp.where(wid < wextra, 1, 0)
        clo = lo * WIN
        chi = (lo + cnt_w) * WIN
        is_last = wid == nw - 1
        chi_eff = jnp.where(is_last, n, chi)
        ones = jnp.full((LANES,), 1, jnp.int32)

        def build_sorted(src_hbm, t):
            pltpu.sync_copy(src_hbm, stage)

            def rbody(vi, cnt):
                off = pl.multiple_of(vi * LANES, LANES)
                r = stage[pl.ds(off, LANES)]
                mine = jnp.logical_and(r >= clo, r < chi_eff)
                pos = cnt + plsc.cumsum(jnp.where(mine, 1, 0)) - 1
                pack = (off + lanes()) + lax.shift_left(r - clo, BBITS)
                plsc.store_scatter(raw, [pos], pack, mask=mine)
                return cnt + plsc.all_reduce_population_count(mine)

            cntv = lax.fori_loop(0, batch // LANES, rbody,
                                 jnp.zeros((LANES,), jnp.int32))
            cnt = lax.reduce_max(cntv, (0,))
            gmax = lax.shift_right_logical(cnt + (LANES - 1), 4)

            for l in range(LANES):
                for c in range(NB // LANES):
                    cnts[l, pl.ds(c * LANES, LANES)] = jnp.zeros(
                        (LANES,), jnp.int32)

            def hbody(g, carry):
                off = g * LANES
                pack = plsc.load_gather(raw, [off + lanes()])
                valid = (off + lanes()) < cnt
                w = lax.shift_right_logical(pack, BBITS + WSHIFT)
                plsc.addupdate_scatter(cnts, [lanes(), w], ones, mask=valid)
                return carry

            lax.fori_loop(0, gmax, hbody, 0)

            gstart = jnp.zeros((LANES,), jnp.int32)
            for c in range(NB // LANES):
                run = jnp.zeros((LANES,), jnp.int32)
                for l in range(LANES):
                    row = cnts[l, pl.ds(c * LANES, LANES)]
                    lpfx[l, pl.ds(c * LANES, LANES)] = run
                    run = run + row
                starts[t, pl.ds(c * LANES, LANES)] = (
                    gstart + plsc.cumsum(run) - run)
                gstart = gstart + lax.reduce_sum(run, (0,))

            for l in range(LANES):
                for c in range(NB // LANES):
                    cnts[l, pl.ds(c * LANES, LANES)] = jnp.zeros(
                        (LANES,), jnp.int32)

            def sbody(g, carry):
                off = g * LANES
                pack = plsc.load_gather(raw, [off + lanes()])
                valid = (off + lanes()) < cnt
                w = lax.shift_right_logical(pack, BBITS + WSHIFT)
                fill = plsc.load_gather(cnts, [lanes(), w])
                sb = plsc.load_gather(starts, [jnp.full((LANES,), t,
                                                        jnp.int32), w])
                lp = plsc.load_gather(lpfx, [lanes(), w])
                pos = sb + lp + fill
                plsc.store_scatter(shits, [jnp.full((LANES,), t, jnp.int32),
                                           pos], pack, mask=valid)
                plsc.addupdate_scatter(cnts, [lanes(), w], ones, mask=valid)
                return carry

            lax.fori_loop(0, gmax, sbody, 0)
            return cnt

        ucnt = build_sorted(users_hbm, 0)
        icnt = build_sorted(items_hbm, 1)

        for p in range(2):
            uridx[p, pl.ds(0, LANES)] = jnp.full((LANES,), batch, jnp.int32)
            iridx[p, pl.ds(0, LANES)] = jnp.full((LANES,), batch, jnp.int32)

        def fire(k):
            c0 = pl.multiple_of((lo + k) * WIN, 128)
            buf = lax.rem(k, 2)
            pltpu.async_copy(ut_hbm.at[:, pl.ds(c0, WIN)], uwin.at[buf],
                             wsem.at[buf])
            pltpu.async_copy(it_hbm.at[:, pl.ds(c0, WIN)], iwin.at[buf],
                             wsem.at[buf])

        fire(0)

        def win_bounds(t, k):
            vec = plsc.load_gather(
                starts, [jnp.full((LANES,), t, jnp.int32),
                         k + jnp.minimum(lanes(), 1)])
            s0 = lax.reduce_max(jnp.where(lanes() == 0, vec, 0), (0,))
            s1 = lax.reduce_max(jnp.where(lanes() == 1, vec, 0), (0,))
            return s0, s1

        def extract_table(win_ref, t, cnt, ring, ridx, rows_hbm, fsem,
                          k, par, rpos, pend):
            s0, s1 = win_bounds(t, k)
            gmax = lax.shift_right_logical(s1 - s0 + (LANES - 1), 4)

            def gbody(g, carry):
                par, rpos, pend = carry
                off = s0 + g * LANES
                pack = plsc.load_gather(
                    shits, [jnp.full((LANES,), t, jnp.int32), off + lanes()])
                inwin = (off + lanes()) < s1
                nhit = plsc.all_reduce_population_count(inwin)

                need_flush = lax.reduce_max(rpos, (0,)) > RING - LANES

                def flush(par, rpos, pend):
                    @pl.when(pend > 0)
                    def _():
                        pltpu.make_async_copy(
                            ring.at[lax.rem(par + 1, 2)],
                            rows_hbm.at[pl.ds(0, RING)], fsem).wait()
                    pltpu.async_copy(
                        ring.at[par], rows_hbm.at[ridx.at[par]], fsem)
                    newpar = lax.rem(par + 1, 2)
                    ridx[newpar, pl.ds(0, LANES)] = jnp.full(
                        (LANES,), batch, jnp.int32)
                    return newpar, jnp.zeros((LANES,), jnp.int32), pend + 1

                par, rpos, pend = lax.cond(
                    need_flush, flush, lambda a, b, c: (a, b, c),
                    par, rpos, pend)

                b = lax.bitwise_and(pack, (1 << BBITS) - 1)
                col = lax.bitwise_and(
                    lax.shift_right_logical(pack, BBITS), WIN - 1)
                slot = rpos + plsc.cumsum(jnp.where(inwin, 1, 0)) - 1
                pvec = jnp.full((LANES,), 0, jnp.int32) + par
                for d in range(EMBED):
                    dvec = jnp.full((LANES,), d, jnp.int32)
                    vals = plsc.load_gather(
                        win_ref, [dvec, jnp.where(inwin, col, 0)])
                    plsc.store_scatter(ring, [pvec, slot, dvec], vals,
                                       mask=inwin)
                plsc.store_scatter(ridx, [pvec, slot], b, mask=inwin)
                return par, rpos + nhit, pend

            return lax.fori_loop(0, gmax, gbody, (par, rpos, pend))

        def wloop(k, carry):
            up, ur, upend, ip, ir, ipend = carry
            cur = lax.rem(k, 2)

            @pl.when(k + 1 < cnt_w)
            def _():
                fire(k + 1)

            pltpu.make_async_copy(ut_hbm.at[:, pl.ds(0, WIN)], uwin.at[cur],
                                  wsem.at[cur]).wait()
            pltpu.make_async_copy(it_hbm.at[:, pl.ds(0, WIN)], iwin.at[cur],
                                  wsem.at[cur]).wait()

            up, ur, upend = extract_table(
                uwin.at[cur], 0, ucnt, uring, uridx, urows_hbm, fsemu,
                k, up, ur, upend)
            ip, ir, ipend = extract_table(
                iwin.at[cur], 1, icnt, iring, iridx, irows_hbm, fsemi,
                k, ip, ir, ipend)
            return up, ur, upend, ip, ir, ipend

        zero = jnp.zeros((LANES,), jnp.int32)
        carry = lax.fori_loop(0, cnt_w, wloop, (0, zero, 0, 0, zero, 0))
        up, ur, upend, ip, ir, ipend = carry

        if tail:
            def do_tail(up, ur, upend, ip, ir, ipend):
                c0 = nfull * WIN
                pltpu.sync_copy(ut_hbm.at[:, pl.ds(c0, tail)], utail)
                pltpu.sync_copy(it_hbm.at[:, pl.ds(c0, tail)], itail)
                up, ur, upend = extract_table(
                    utail, 0, ucnt, uring, uridx, urows_hbm, fsemu,
                    cnt_w, up, ur, upend)
                ip, ir, ipend = extract_table(
                    itail, 1, icnt, iring, iridx, irows_hbm, fsemi,
                    cnt_w, ip, ir, ipend)
                return up, ur, upend, ip, ir, ipend

            up, ur, upend, ip, ir, ipend = lax.cond(
                is_last, do_tail,
                lambda a, b, c, d, e, f: (a, b, c, d, e, f),
                up, ur, upend, ip, ir, ipend)

        def finalize(ring, ridx, rows_hbm, fsem, par, rpos, pend):
            has_rows = lax.reduce_max(rpos, (0,)) > 0

            @pl.when(jnp.logical_and(has_rows, pend > 0))
            def _():
                pltpu.make_async_copy(ring.at[lax.rem(par + 1, 2)],
                                      rows_hbm.at[pl.ds(0, RING)],
                                      fsem).wait()

            @pl.when(has_rows)
            def _():
                pltpu.async_copy(ring.at[par], rows_hbm.at[ridx.at[par]],
                                 fsem)

            pend = jnp.where(has_rows, 1, jnp.minimum(pend, 1))

            @pl.when(pend > 0)
            def _():
                pltpu.make_async_copy(ring.at[0],
                                      rows_hbm.at[pl.ds(0, RING)],
                                      fsem).wait()

        finalize(uring, uridx, urows_hbm, fsemu, up, ur, upend)
        finalize(iring, iridx, irows_hbm, fsemi, ip, ir, ipend)

    CH = 128

    @functools.partial(
        pl.kernel,
        mesh=mesh,
        compiler_params=pltpu.CompilerParams(needs_layout_passes=False),
        out_type=jax.ShapeDtypeStruct((batch,), jnp.float32),
        scratch_types=[
            pltpu.VMEM((CH, ROWPAD), jnp.float32),
            pltpu.VMEM((CH, ROWPAD), jnp.float32),
            pltpu.VMEM((bpw,), jnp.float32),
        ],
    )
    def dots(urows_hbm, irows_hbm, out_hbm, uch, ich, outv):
        wid = lax.axis_index("s") * nc + lax.axis_index("c")
        base = wid * bpw

        def chunk(j, carry):
            row0 = pl.multiple_of(base + j * CH, 8)
            pltpu.sync_copy(urows_hbm.at[pl.ds(row0, CH)], uch)
            pltpu.sync_copy(irows_hbm.at[pl.ds(row0, CH)], ich)

            def group(g, c2):
                rvec = g * LANES + lanes()
                acc = jnp.zeros((LANES,), jnp.float32)
                for d in range(EMBED):
                    dvec = jnp.full((LANES,), d, jnp.int32)
                    uv = plsc.load_gather(uch, [rvec, dvec])
                    iv = plsc.load_gather(ich, [rvec, dvec])
                    acc = acc + uv * iv
                off = pl.multiple_of(j * CH + g * LANES, LANES)
                outv[pl.ds(off, LANES)] = acc
                return c2

            lax.fori_loop(0, CH // LANES, group, 0)
            return carry

        lax.fori_loop(0, bpw // CH, chunk, 0)
        pltpu.sync_copy(outv, out_hbm.at[pl.ds(base, bpw)])

    urows, irows = scan(users, items, ut, it)
    return dots(urows, irows)


# speedup vs baseline: 2.7008x; 2.7008x over previous
import functools

import jax
import jax.numpy as jnp
from jax import lax
from jax.experimental import pallas as pl
from jax.experimental.pallas import tpu as pltpu
from jax.experimental.pallas import tpu_sc as plsc

EMBED = 32
LANES = 16
WIN = 256
WSHIFT = 8
RING = 32
ROWPAD = 128
NB = 128
BBITS = 14


def kernel(users, items, user_table, item_table):
    users = users.astype(jnp.int32)
    items = items.astype(jnp.int32)
    ut = user_table.T
    it = item_table.T
    batch = users.shape[0]
    n = user_table.shape[0]

    nfull = n // WIN
    tail = n - nfull * WIN

    info = plsc.get_sparse_core_info()
    nc, ns = info.num_cores, info.num_subcores
    nw = nc * ns
    bpw = batch // nw
    wper = nfull // nw
    wextra = nfull - wper * nw
    brows = batch + 8

    mesh = plsc.VectorSubcoreMesh(core_axis_name="c", subcore_axis_name="s")
    lanes = lambda: lax.iota(jnp.int32, LANES)

    @functools.partial(
        pl.kernel,
        mesh=mesh,
        compiler_params=pltpu.CompilerParams(needs_layout_passes=False),
        out_type=(jax.ShapeDtypeStruct((brows, ROWPAD), jnp.float32),
                  jax.ShapeDtypeStruct((brows, ROWPAD), jnp.float32)),
        scratch_types=[
            pltpu.VMEM((batch,), jnp.int32),
            pltpu.VMEM((batch,), jnp.int32),
            pltpu.VMEM((2, batch), jnp.int32),
            pltpu.VMEM((2, NB), jnp.int32),
            pltpu.VMEM((LANES, NB), jnp.int32),
            pltpu.VMEM((LANES, NB), jnp.int32),
            pltpu.VMEM((2, EMBED, WIN), jnp.float32),
            pltpu.VMEM((2, EMBED, WIN), jnp.float32),
            pltpu.VMEM((2, RING, ROWPAD), jnp.float32),
            pltpu.VMEM((2, RING, ROWPAD), jnp.float32),
            pltpu.VMEM((2, RING), jnp.int32),
            pltpu.VMEM((2, RING), jnp.int32),
            pltpu.VMEM((EMBED, tail or 1), jnp.float32),
            pltpu.VMEM((EMBED, tail or 1), jnp.float32),
            pltpu.SMEM((2, NB), jnp.int32),
            pltpu.SemaphoreType.DMA((2,)),
            pltpu.SemaphoreType.DMA,
            pltpu.SemaphoreType.DMA,
        ],
    )
    def scan(users_hbm, items_hbm, ut_hbm, it_hbm, urows_hbm, irows_hbm,
             stage, raw, shits, starts, cnts, lpfx, uwin, iwin, uring, iring,
             uridx, iridx, utail, itail, sbnd, wsem, fsemu, fsemi):
        wid = lax.axis_index("s") * nc + lax.axis_index("c")
        lo = wid * wper + jnp.minimum(wid, wextra)
        cnt_w = wper + jnp.where(wid < wextra, 1, 0)
        clo = lo * WIN
        chi = (lo + cnt_w) * WIN
        is_last = wid == nw - 1
        chi_eff = jnp.where(is_last, n, chi)
        ones = jnp.full((LANES,), 1, jnp.int32)

        def build_sorted(src_hbm, t):
            pltpu.sync_copy(src_hbm, stage)

            def rbody(vi, cnt):
                off = pl.multiple_of(vi * LANES, LANES)
                r = stage[pl.ds(off, LANES)]
                mine = jnp.logical_and(r >= clo, r < chi_eff)
                pos = cnt + plsc.cumsum(jnp.where(mine, 1, 0)) - 1
                pack = (off + lanes()) + lax.shift_left(r - clo, BBITS)
                plsc.store_scatter(raw, [pos], pack, mask=mine)
                return cnt + plsc.all_reduce_population_count(mine)

            cntv = lax.fori_loop(0, batch // LANES, rbody,
                                 jnp.zeros((LANES,), jnp.int32))
            cnt = lax.reduce_max(cntv, (0,))
            gmax = lax.shift_right_logical(cnt + (LANES - 1), 4)

            for l in range(LANES):
                for c in range(NB // LANES):
                    cnts[l, pl.ds(c * LANES, LANES)] = jnp.zeros(
                        (LANES,), jnp.int32)

            def hbody(g, carry):
                off = g * LANES
                pack = plsc.load_gather(raw, [off + lanes()])
                valid = (off + lanes()) < cnt
                w = lax.shift_right_logical(pack, BBITS + WSHIFT)
                plsc.addupdate_scatter(cnts, [lanes(), w], ones, mask=valid)
                return carry

            lax.fori_loop(0, gmax, hbody, 0)

            gstart = jnp.zeros((LANES,), jnp.int32)
            for c in range(NB // LANES):
                run = jnp.zeros((LANES,), jnp.int32)
                for l in range(LANES):
                    row = cnts[l, pl.ds(c * LANES, LANES)]
                    lpfx[l, pl.ds(c * LANES, LANES)] = run
                    run = run + row
                svec = gstart + plsc.cumsum(run) - run
                starts[t, pl.ds(c * LANES, LANES)] = svec
                for l in range(LANES):
                    sbnd[t, c * LANES + l] = lax.reduce_max(
                        jnp.where(lanes() == l, svec, 0), (0,))
                gstart = gstart + lax.reduce_sum(run, (0,))

            for l in range(LANES):
                for c in range(NB // LANES):
                    cnts[l, pl.ds(c * LANES, LANES)] = jnp.zeros(
                        (LANES,), jnp.int32)

            def sbody(g, carry):
                off = g * LANES
                pack = plsc.load_gather(raw, [off + lanes()])
                valid = (off + lanes()) < cnt
                w = lax.shift_right_logical(pack, BBITS + WSHIFT)
                fill = plsc.load_gather(cnts, [lanes(), w])
                sb = plsc.load_gather(starts, [jnp.full((LANES,), t,
                                                        jnp.int32), w])
                lp = plsc.load_gather(lpfx, [lanes(), w])
                pos = sb + lp + fill
                plsc.store_scatter(shits, [jnp.full((LANES,), t, jnp.int32),
                                           pos], pack, mask=valid)
                plsc.addupdate_scatter(cnts, [lanes(), w], ones, mask=valid)
                return carry

            lax.fori_loop(0, gmax, sbody, 0)
            return cnt

        ucnt = build_sorted(users_hbm, 0)
        icnt = build_sorted(items_hbm, 1)

        for p in range(2):
            for q in range(RING // LANES):
                uridx[p, pl.ds(q * LANES, LANES)] = jnp.full(
                    (LANES,), batch, jnp.int32)
                iridx[p, pl.ds(q * LANES, LANES)] = jnp.full(
                    (LANES,), batch, jnp.int32)

        def fire(k):
            c0 = pl.multiple_of((lo + k) * WIN, 128)
            buf = lax.rem(k, 2)
            pltpu.async_copy(ut_hbm.at[:, pl.ds(c0, WIN)], uwin.at[buf],
                             wsem.at[buf])
            pltpu.async_copy(it_hbm.at[:, pl.ds(c0, WIN)], iwin.at[buf],
                             wsem.at[buf])

        fire(0)

        def win_bounds(t, k):
            return sbnd[t, k], sbnd[t, k + 1]

        def extract_table(win_ref, t, cnt, ring, ridx, rows_hbm, fsem,
                          k, par, rpos, pend):
            s0, s1 = win_bounds(t, k)
            gmax = lax.shift_right_logical(s1 - s0 + (LANES - 1), 4)

            def gbody(g, carry):
                par, rpos, pend = carry
                off = s0 + g * LANES
                pack = plsc.load_gather(
                    shits, [jnp.full((LANES,), t, jnp.int32), off + lanes()])
                inwin = (off + lanes()) < s1
                nhit = jnp.minimum(s1 - off, LANES)

                need_flush = rpos > RING - LANES

                def flush(par, rpos, pend):
                    @pl.when(pend > 0)
                    def _():
                        pltpu.make_async_copy(
                            ring.at[lax.rem(par + 1, 2)],
                            rows_hbm.at[pl.ds(0, RING)], fsem).wait()
                    pltpu.async_copy(
                        ring.at[par], rows_hbm.at[ridx.at[par]], fsem)
                    newpar = lax.rem(par + 1, 2)
                    for q in range(RING // LANES):
                        ridx[newpar, pl.ds(q * LANES, LANES)] = jnp.full(
                            (LANES,), batch, jnp.int32)
                    return newpar, 0 * par, pend + 1

                par, rpos, pend = lax.cond(
                    need_flush, flush, lambda a, b, c: (a, b, c),
                    par, rpos, pend)

                b = lax.bitwise_and(pack, (1 << BBITS) - 1)
                col = lax.bitwise_and(
                    lax.shift_right_logical(pack, BBITS), WIN - 1)
                slot = rpos + lanes()
                pvec = jnp.full((LANES,), 0, jnp.int32) + par
                for d in range(EMBED):
                    dvec = jnp.full((LANES,), d, jnp.int32)
                    vals = plsc.load_gather(
                        win_ref, [dvec, jnp.where(inwin, col, 0)])
                    plsc.store_scatter(ring, [pvec, slot, dvec], vals,
                                       mask=inwin)
                plsc.store_scatter(ridx, [pvec, slot], b, mask=inwin)
                return par, rpos + nhit, pend

            return lax.fori_loop(0, gmax, gbody, (par, rpos, pend))

        def wloop(k, carry):
            up, ur, upend, ip, ir, ipend = carry
            cur = lax.rem(k, 2)

            @pl.when(k + 1 < cnt_w)
            def _():
                fire(k + 1)

            pltpu.make_async_copy(ut_hbm.at[:, pl.ds(0, WIN)], uwin.at[cur],
                                  wsem.at[cur]).wait()
            pltpu.make_async_copy(it_hbm.at[:, pl.ds(0, WIN)], iwin.at[cur],
                                  wsem.at[cur]).wait()

            up, ur, upend = extract_table(
                uwin.at[cur], 0, ucnt, uring, uridx, urows_hbm, fsemu,
                k, up, ur, upend)
            ip, ir, ipend = extract_table(
                iwin.at[cur], 1, icnt, iring, iridx, irows_hbm, fsemi,
                k, ip, ir, ipend)
            return up, ur, upend, ip, ir, ipend

        carry = lax.fori_loop(0, cnt_w, wloop, (0, 0, 0, 0, 0, 0))
        up, ur, upend, ip, ir, ipend = carry

        if tail:
            def do_tail(up, ur, upend, ip, ir, ipend):
                c0 = nfull * WIN
                pltpu.sync_copy(ut_hbm.at[:, pl.ds(c0, tail)], utail)
                pltpu.sync_copy(it_hbm.at[:, pl.ds(c0, tail)], itail)
                up, ur, upend = extract_table(
                    utail, 0, ucnt, uring, uridx, urows_hbm, fsemu,
                    cnt_w, up, ur, upend)
                ip, ir, ipend = extract_table(
                    itail, 1, icnt, iring, iridx, irows_hbm, fsemi,
                    cnt_w, ip, ir, ipend)
                return up, ur, upend, ip, ir, ipend

            up, ur, upend, ip, ir, ipend = lax.cond(
                is_last, do_tail,
                lambda a, b, c, d, e, f: (a, b, c, d, e, f),
                up, ur, upend, ip, ir, ipend)

        def finalize(ring, ridx, rows_hbm, fsem, par, rpos, pend):
            has_rows = rpos > 0

            @pl.when(jnp.logical_and(has_rows, pend > 0))
            def _():
                pltpu.make_async_copy(ring.at[lax.rem(par + 1, 2)],
                                      rows_hbm.at[pl.ds(0, RING)],
                                      fsem).wait()

            @pl.when(has_rows)
            def _():
                pltpu.async_copy(ring.at[par], rows_hbm.at[ridx.at[par]],
                                 fsem)

            pend = jnp.where(has_rows, 1, jnp.minimum(pend, 1))

            @pl.when(pend > 0)
            def _():
                pltpu.make_async_copy(ring.at[0],
                                      rows_hbm.at[pl.ds(0, RING)],
                                      fsem).wait()

        finalize(uring, uridx, urows_hbm, fsemu, up, ur, upend)
        finalize(iring, iridx, irows_hbm, fsemi, ip, ir, ipend)

    CH = 128

    @functools.partial(
        pl.kernel,
        mesh=mesh,
        compiler_params=pltpu.CompilerParams(needs_layout_passes=False),
        out_type=jax.ShapeDtypeStruct((batch,), jnp.float32),
        scratch_types=[
            pltpu.VMEM((CH, ROWPAD), jnp.float32),
            pltpu.VMEM((CH, ROWPAD), jnp.float32),
            pltpu.VMEM((bpw,), jnp.float32),
        ],
    )
    def dots(urows_hbm, irows_hbm, out_hbm, uch, ich, outv):
        wid = lax.axis_index("s") * nc + lax.axis_index("c")
        base = wid * bpw

        def chunk(j, carry):
            row0 = pl.multiple_of(base + j * CH, 8)
            pltpu.sync_copy(urows_hbm.at[pl.ds(row0, CH)], uch)
            pltpu.sync_copy(irows_hbm.at[pl.ds(row0, CH)], ich)

            def group(g, c2):
                rvec = g * LANES + lanes()
                acc = jnp.zeros((LANES,), jnp.float32)
                for d in range(EMBED):
                    dvec = jnp.full((LANES,), d, jnp.int32)
                    uv = plsc.load_gather(uch, [rvec, dvec])
                    iv = plsc.load_gather(ich, [rvec, dvec])
                    acc = acc + uv * iv
                off = pl.multiple_of(j * CH + g * LANES, LANES)
                outv[pl.ds(off, LANES)] = acc
                return c2

            lax.fori_loop(0, CH // LANES, group, 0)
            return carry

        lax.fori_loop(0, bpw // CH, chunk, 0)
        pltpu.sync_copy(outv, out_hbm.at[pl.ds(base, bpw)])

    urows, irows = scan(users, items, ut, it)
    return dots(urows, irows)
